# Initial kernel scaffold; baseline (speedup 1.0000x reference)
import jax
import jax.numpy as jnp
from jax.experimental import pallas as pl


def kernel(token_ids, weight):
    raise NotImplementedError("write your pallas kernel here")


# speedup vs baseline: 1.7702x; 1.7702x over previous
import jax
import jax.numpy as jnp
from jax import lax
from jax.experimental import pallas as pl
from jax.experimental.pallas import tpu as pltpu
from jax.experimental.pallas import tpu_sc as plsc

D_MODEL = 64
NUM_CORES = 2
NUM_SUBCORES = 16
NUM_WORKERS = NUM_CORES * NUM_SUBCORES
CHUNK = 128
NBUF = 2


def _gather_body(ids_hbm, table_hbm, out_hbm, idx_v, rows_v, gsems):
    wid = lax.axis_index("s") * NUM_CORES + lax.axis_index("c")
    b_total = ids_hbm.shape[0]
    b_per_w = b_total // NUM_WORKERS
    nchunks = b_per_w // CHUNK
    base = wid * b_per_w

    def start_gather(b, g):
        off = pl.multiple_of(base + g * CHUNK, 8)
        pltpu.sync_copy(ids_hbm.at[pl.ds(off, CHUNK)], idx_v.at[b])
        pltpu.async_copy(table_hbm.at[idx_v.at[b]], rows_v.at[b], gsems.at[b])

    for b in range(NBUF):
        start_gather(b, b)

    @pl.loop(0, nchunks, step=NBUF)
    def _(g0):
        for b in range(NBUF):
            g = g0 + b
            pltpu.make_async_copy(
                table_hbm.at[idx_v.at[b]], rows_v.at[b], gsems.at[b]
            ).wait()
            off = pl.multiple_of(base + g * CHUNK, 8)
            pltpu.sync_copy(rows_v.at[b], out_hbm.at[pl.ds(off, CHUNK)])

            @pl.when(g + NBUF < nchunks)
            def _():
                start_gather(b, g + NBUF)


def kernel(token_ids, weight):
    n_tok, seq = token_ids.shape
    b_total = n_tok * seq
    flat_ids = token_ids.reshape(b_total).astype(jnp.int32)

    mesh = plsc.VectorSubcoreMesh(core_axis_name="c", subcore_axis_name="s")
    out = pl.kernel(
        _gather_body,
        out_type=jax.ShapeDtypeStruct((b_total, D_MODEL), jnp.float32),
        mesh=mesh,
        scratch_types=[
            pltpu.VMEM((NBUF, CHUNK), jnp.int32),
            pltpu.VMEM((NBUF, CHUNK, D_MODEL), jnp.float32),
            pltpu.SemaphoreType.DMA((NBUF,)),
        ],
        compiler_params=pltpu.CompilerParams(use_tc_tiling_on_sc=False),
    )(flat_ids, weight)
    return out.reshape(n_tok, seq, D_MODEL)


# speedup vs baseline: 1.8739x; 1.0586x over previous
import jax
import jax.numpy as jnp
from jax import lax
from jax.experimental import pallas as pl
from jax.experimental.pallas import tpu as pltpu
from jax.experimental.pallas import tpu_sc as plsc

D_MODEL = 64
NUM_CORES = 2
NUM_SUBCORES = 16
NUM_WORKERS = NUM_CORES * NUM_SUBCORES
CHUNK = 128
NBUF = 8


def _gather_body(ids_hbm, table_hbm, out_hbm, idx_all, rows_v, gsems, osems):
    wid = lax.axis_index("s") * NUM_CORES + lax.axis_index("c")
    nchunks = idx_all.shape[0]
    b_per_w = nchunks * CHUNK
    base = wid * b_per_w

    pltpu.sync_copy(ids_hbm.at[wid], idx_all)

    def start_gather(b, g):
        pltpu.async_copy(table_hbm.at[idx_all.at[g]], rows_v.at[b], gsems.at[b])

    for b in range(NBUF):
        start_gather(b, b)

    @pl.loop(0, nchunks, step=NBUF)
    def _(g0):
        for b in range(NBUF):
            g = g0 + b
            pltpu.make_async_copy(
                table_hbm.at[idx_all.at[g]], rows_v.at[b], gsems.at[b]
            ).wait()
            off = pl.multiple_of(base + g * CHUNK, 8)
            out_slice = out_hbm.at[pl.ds(off, CHUNK)]
            pltpu.async_copy(rows_v.at[b], out_slice, osems.at[b])

            @pl.when(g + NBUF < nchunks)
            def _():
                pltpu.make_async_copy(rows_v.at[b], out_slice, osems.at[b]).wait()
                start_gather(b, g + NBUF)

    for b in range(NBUF):
        g = nchunks - NBUF + b
        off = pl.multiple_of(base + g * CHUNK, 8)
        pltpu.make_async_copy(
            rows_v.at[b], out_hbm.at[pl.ds(off, CHUNK)], osems.at[b]
        ).wait()


def kernel(token_ids, weight):
    n_tok, seq = token_ids.shape
    b_total = n_tok * seq
    b_per_w = b_total // NUM_WORKERS
    nchunks = b_per_w // CHUNK
    ids3d = token_ids.reshape(NUM_WORKERS, nchunks, CHUNK).astype(jnp.int32)

    mesh = plsc.VectorSubcoreMesh(core_axis_name="c", subcore_axis_name="s")
    out = pl.kernel(
        _gather_body,
        out_type=jax.ShapeDtypeStruct((b_total, D_MODEL), jnp.float32),
        mesh=mesh,
        scratch_types=[
            pltpu.VMEM((nchunks, CHUNK), jnp.int32),
            pltpu.VMEM((NBUF, CHUNK, D_MODEL), jnp.float32),
            pltpu.SemaphoreType.DMA((NBUF,)),
            pltpu.SemaphoreType.DMA((NBUF,)),
        ],
        compiler_params=pltpu.CompilerParams(use_tc_tiling_on_sc=False),
    )(ids3d, weight)
    return out.reshape(n_tok, seq, D_MODEL)


# speedup vs baseline: 1.8759x; 1.0011x over previous
import jax
import jax.numpy as jnp
from jax import lax
from jax.experimental import pallas as pl
from jax.experimental.pallas import tpu as pltpu
from jax.experimental.pallas import tpu_sc as plsc

D_MODEL = 64
NUM_CORES = 2
NUM_SUBCORES = 16
NUM_WORKERS = NUM_CORES * NUM_SUBCORES
CHUNK = 128
NBUF = 8


def _gather_body(ids_hbm, table_hbm, out_hbm, idx_all, rows_v, gsems, osems):
    wid = lax.axis_index("s") * NUM_CORES + lax.axis_index("c")
    b_per_w = idx_all.shape[0]
    nchunks = b_per_w // CHUNK
    base = wid * b_per_w

    pltpu.sync_copy(ids_hbm.at[pl.ds(pl.multiple_of(base, 8), b_per_w)], idx_all)

    def start_gather(b, g):
        idx = idx_all.at[pl.ds(g * CHUNK, CHUNK)]
        pltpu.async_copy(table_hbm.at[idx], rows_v.at[b], gsems.at[b])

    for b in range(NBUF):
        start_gather(b, b)

    @pl.loop(0, nchunks, step=NBUF)
    def _(g0):
        for b in range(NBUF):
            g = g0 + b
            idx = idx_all.at[pl.ds(g * CHUNK, CHUNK)]
            pltpu.make_async_copy(
                table_hbm.at[idx], rows_v.at[b], gsems.at[b]
            ).wait()
            off = pl.multiple_of(base + g * CHUNK, 8)
            out_slice = out_hbm.at[pl.ds(off, CHUNK)]
            pltpu.async_copy(rows_v.at[b], out_slice, osems.at[b])

            @pl.when(g + NBUF < nchunks)
            def _():
                pltpu.make_async_copy(rows_v.at[b], out_slice, osems.at[b]).wait()
                start_gather(b, g + NBUF)

    for b in range(NBUF):
        g = nchunks - NBUF + b
        off = pl.multiple_of(base + g * CHUNK, 8)
        pltpu.make_async_copy(
            rows_v.at[b], out_hbm.at[pl.ds(off, CHUNK)], osems.at[b]
        ).wait()


def kernel(token_ids, weight):
    n_tok, seq = token_ids.shape
    b_total = n_tok * seq
    b_per_w = b_total // NUM_WORKERS
    ids_flat = token_ids.reshape(b_total).astype(jnp.int32)

    mesh = plsc.VectorSubcoreMesh(core_axis_name="c", subcore_axis_name="s")
    out = pl.kernel(
        _gather_body,
        out_type=jax.ShapeDtypeStruct((b_total, D_MODEL), jnp.float32),
        mesh=mesh,
        scratch_types=[
            pltpu.VMEM((b_per_w,), jnp.int32),
            pltpu.VMEM((NBUF, CHUNK, D_MODEL), jnp.float32),
            pltpu.SemaphoreType.DMA((NBUF,)),
            pltpu.SemaphoreType.DMA((NBUF,)),
        ],
        compiler_params=pltpu.CompilerParams(use_tc_tiling_on_sc=False),
    )(ids_flat, weight)
    return out.reshape(n_tok, seq, D_MODEL)
